# baseline (speedup 1.0000x reference)
import functools

import jax
import jax.numpy as jnp
from jax import lax
from jax.experimental import pallas as pl
from jax.experimental.pallas import tpu as pltpu
from jax.experimental.pallas import tpu_sc as plsc

N_EXP = 8
BLK = 256


def _router_body(x_ref, wg_ref, e_ref, w_ref, rank_ref, cnt_ref,
                 aux_ref, run_ref, acc_ref, *, n_tb, n_tokens, tb_sz):
    tb = pl.program_id(0)
    xb = x_ref[...]

    logits = jnp.dot(xb.astype(jnp.bfloat16),
                     wg_ref[...].astype(jnp.bfloat16),
                     preferred_element_type=jnp.float32)
    m = jnp.max(logits, axis=-1, keepdims=True)
    ex = jnp.exp(logits - m)
    p = ex / jnp.sum(ex, axis=-1, keepdims=True)
    lane = jax.lax.broadcasted_iota(jnp.int32, p.shape, 1).astype(jnp.float32)
    p1 = jnp.max(p, axis=-1, keepdims=True)
    i1 = jnp.min(jnp.where(p == p1, lane, float(N_EXP)), axis=-1,
                 keepdims=True)
    pm = jnp.where(lane == i1, -1.0, p)
    p2 = jnp.max(pm, axis=-1, keepdims=True)
    i2 = jnp.min(jnp.where(pm == p2, lane, float(N_EXP)), axis=-1,
                 keepdims=True)
    s = p1 + p2
    w1n = p1 / s
    w2n = p2 / s

    cnt_blk = jnp.sum(
        jnp.where(lane == i1, 1.0, 0.0) + jnp.where(lane == i2, 1.0, 0.0),
        axis=0, keepdims=True)
    sp = jnp.sum(p, axis=0, keepdims=True)
    new = jnp.concatenate([cnt_blk, sp], axis=0)

    @pl.when(tb == 0)
    def _():
        acc_ref[...] = new
        run_ref[...] = jnp.zeros_like(run_ref)

    @pl.when(tb > 0)
    def _():
        acc_ref[...] = acc_ref[...] + new

    @pl.when(tb == n_tb - 1)
    def _():
        a = acc_ref[...]
        aux_ref[...] = (N_EXP / (n_tokens * n_tokens)) * jnp.sum(
            a[0:1, :] * a[1:2, :], axis=-1, keepdims=True)

    e_flat = jnp.concatenate([i1, i2], axis=0)
    w_flat = jnp.concatenate([w1n, w2n], axis=0)
    e_ref[...] = e_flat
    w_ref[...] = w_flat

    lane2 = jax.lax.broadcasted_iota(
        jnp.int32, (2 * tb_sz, N_EXP), 1).astype(jnp.float32)
    oh = (lane2 == e_flat).astype(jnp.bfloat16)
    r = jax.lax.broadcasted_iota(jnp.int32, (2 * tb_sz, 2 * tb_sz), 0)
    c = jax.lax.broadcasted_iota(jnp.int32, (2 * tb_sz, 2 * tb_sz), 1)
    ltri = (c < r).astype(jnp.bfloat16)
    rank_in_blk = jnp.dot(ltri, oh, preferred_element_type=jnp.float32)
    rank_e = jnp.sum(rank_in_blk * oh.astype(jnp.float32), axis=-1,
                     keepdims=True)
    base = jnp.sum(oh.astype(jnp.float32) * run_ref[...], axis=-1,
                   keepdims=True)
    rank_ref[...] = base + rank_e
    run_ref[...] = run_ref[...] + cnt_blk

    @pl.when(tb == n_tb - 1)
    def _():
        cnt = run_ref[...]
        cap = jnp.ceil(cnt / BLK) * BLK
        r8 = jax.lax.broadcasted_iota(jnp.int32, (N_EXP, N_EXP), 0)
        c8 = jax.lax.broadcasted_iota(jnp.int32, (N_EXP, N_EXP), 1)
        ut = (r8 <= c8).astype(jnp.float32)
        cum = jnp.dot(cap, ut, preferred_element_type=jnp.float32)
        cnt_ref[...] = jnp.concatenate([cum - cap, cum], axis=1)


def _run_router(x_flat, Wg, n_tokens, C):
    TB = 1024
    n_tb = n_tokens // TB
    body = functools.partial(_router_body, n_tb=n_tb, n_tokens=n_tokens,
                             tb_sz=TB)
    return pl.pallas_call(
        body,
        grid=(n_tb,),
        in_specs=[
            pl.BlockSpec((TB, C), lambda tb: (tb, 0)),
            pl.BlockSpec((C, N_EXP), lambda tb: (0, 0)),
        ],
        out_specs=[
            pl.BlockSpec((2 * TB, 1), lambda tb: (tb, 0)),
            pl.BlockSpec((2 * TB, 1), lambda tb: (tb, 0)),
            pl.BlockSpec((2 * TB, 1), lambda tb: (tb, 0)),
            pl.BlockSpec((1, 2 * N_EXP), lambda tb: (0, 0)),
            pl.BlockSpec((1, 1), lambda tb: (0, 0)),
        ],
        out_shape=[
            jax.ShapeDtypeStruct((2 * n_tokens, 1), jnp.float32),
            jax.ShapeDtypeStruct((2 * n_tokens, 1), jnp.float32),
            jax.ShapeDtypeStruct((2 * n_tokens, 1), jnp.float32),
            jax.ShapeDtypeStruct((1, 2 * N_EXP), jnp.float32),
            jax.ShapeDtypeStruct((1, 1), jnp.float32),
        ],
        scratch_shapes=[
            pltpu.VMEM((1, N_EXP), jnp.float32),
            pltpu.VMEM((2, N_EXP), jnp.float32),
        ],
        compiler_params=pltpu.CompilerParams(
            dimension_semantics=("arbitrary",),
        ),
    )(x_flat, Wg)


def _finalize_body(e_in_ref, rank_ref, segend_ref, dest_ref, blk_ref,
                   *, nbr, nblk_pad):
    tb = pl.program_id(0)
    seg_f = segend_ref[...]
    e = e_in_ref[...]
    lane = jax.lax.broadcasted_iota(
        jnp.int32, (e.shape[0], N_EXP), 1).astype(jnp.float32)
    oh = (lane == e).astype(jnp.float32)
    start = jnp.sum(oh * seg_f[:, :N_EXP], axis=-1, keepdims=True)
    dest_ref[...] = start + rank_ref[...]

    @pl.when(tb == 0)
    def _():
        ends = seg_f[:, N_EXP:]
        bv = jax.lax.broadcasted_iota(
            jnp.int32, (nblk_pad, N_EXP), 0).astype(jnp.float32)
        acc = jnp.sum((bv * BLK >= ends).astype(jnp.int32),
                      axis=-1, keepdims=True)
        blk_ref[...] = jnp.where(
            jax.lax.broadcasted_iota(jnp.int32, (nblk_pad, 1), 0) >= nbr,
            N_EXP, acc)


def _run_finalize(e_arr, rank, segend, n_asn, nbr, nblk_pad):
    AB = 2048
    return pl.pallas_call(
        functools.partial(_finalize_body, nbr=nbr, nblk_pad=nblk_pad),
        grid=(n_asn // AB,),
        in_specs=[
            pl.BlockSpec((AB, 1), lambda tb: (tb, 0)),
            pl.BlockSpec((AB, 1), lambda tb: (tb, 0)),
            pl.BlockSpec((1, 2 * N_EXP), lambda tb: (0, 0)),
        ],
        out_specs=[
            pl.BlockSpec((AB, 1), lambda tb: (tb, 0)),
            pl.BlockSpec((nblk_pad, 1), lambda tb: (0, 0)),
        ],
        out_shape=[
            jax.ShapeDtypeStruct((n_asn, 1), jnp.float32),
            jax.ShapeDtypeStruct((nblk_pad, 1), jnp.int32),
        ],
        compiler_params=pltpu.CompilerParams(
            dimension_semantics=("arbitrary",),
        ),
    )(e_arr, rank, segend)


def _make_dispatch(n_tokens, C, routed_pad):
    n_asn = 2 * n_tokens
    info = plsc.get_sparse_core_info()
    nw = info.num_cores * info.num_subcores
    a_per_w = n_asn // nw
    n_ch = a_per_w // 64
    mesh = plsc.VectorSubcoreMesh(core_axis_name="c", subcore_axis_name="s")

    @functools.partial(
        pl.kernel, mesh=mesh,
        out_type=jax.ShapeDtypeStruct((routed_pad, C), jnp.float32),
        scratch_types=[
            pltpu.VMEM((a_per_w,), jnp.float32),
            pltpu.VMEM((64,), jnp.int32),
            pltpu.VMEM((64,), jnp.int32),
            pltpu.VMEM((64, C), jnp.float32),
            pltpu.SemaphoreType.DMA,
        ],
    )
    def dispatch(xf32, dest, xg, dest_v, tok_i, dst_i, rows, sem):
        wid = lax.axis_index("s") * info.num_cores + lax.axis_index("c")
        abase = pl.multiple_of(wid * a_per_w, 64)
        l16 = lax.iota(jnp.int32, 16)

        pltpu.sync_copy(dest.at[pl.ds(abase, a_per_w)], dest_v)
        for ch in range(n_ch):
            for j in range(4):
                a_v = abase + ch * 64 + j * 16 + l16
                t_v = ((a_v >> 11) << 10) + (a_v & 1023)
                tok_i[pl.ds(j * 16, 16)] = t_v
                dst_i[pl.ds(j * 16, 16)] = dest_v[
                    pl.ds(ch * 64 + j * 16, 16)].astype(jnp.int32)
            pltpu.async_copy(xf32.at[tok_i], rows, sem).wait()
            pltpu.async_copy(rows, xg.at[dst_i], sem).wait()

    return dispatch


def _gmm_body(s_ref, xg_ref, xs_ref, w1_ref, w2_ref, w3_ref, yg_ref, *, nbr):
    b = pl.program_id(0)
    xb = jnp.where(b < nbr, xg_ref[...], xs_ref[...]).astype(jnp.bfloat16)
    g = jnp.dot(xb, w1_ref[0], preferred_element_type=jnp.float32)
    u = jnp.dot(xb, w2_ref[0], preferred_element_type=jnp.float32)
    act = (g * jax.nn.sigmoid(g) * u).astype(jnp.bfloat16)
    yg_ref[...] = jnp.dot(act, w3_ref[0], preferred_element_type=jnp.float32)


def _run_gmm(blk_e, xg, xb16, w1, w2, w3, nbr, nblk, C, Hp):
    grid_spec = pltpu.PrefetchScalarGridSpec(
        num_scalar_prefetch=1,
        grid=(nblk,),
        in_specs=[
            pl.BlockSpec((BLK, C), lambda b, s: (jnp.minimum(b, nbr - 1), 0)),
            pl.BlockSpec((BLK, C), lambda b, s: (jnp.maximum(b - nbr, 0), 0)),
            pl.BlockSpec((1, C, Hp), lambda b, s: (s[b], 0, 0)),
            pl.BlockSpec((1, C, Hp), lambda b, s: (s[b], 0, 0)),
            pl.BlockSpec((1, Hp, C), lambda b, s: (s[b], 0, 0)),
        ],
        out_specs=pl.BlockSpec((BLK, C), lambda b, s: (b, 0)),
    )
    return pl.pallas_call(
        functools.partial(_gmm_body, nbr=nbr),
        grid_spec=grid_spec,
        out_shape=jax.ShapeDtypeStruct((nblk * BLK, C), jnp.float32),
        compiler_params=pltpu.CompilerParams(
            dimension_semantics=("arbitrary",),
        ),
    )(blk_e, xg, xb16, w1, w2, w3)


def _make_combine(n_tokens, C, routed_pad):
    info = plsc.get_sparse_core_info()
    nw = info.num_cores * info.num_subcores
    t_per_w = n_tokens // nw
    n_ch = t_per_w // 16
    mesh = plsc.VectorSubcoreMesh(core_axis_name="c", subcore_axis_name="s")

    @functools.partial(
        pl.kernel, mesh=mesh,
        out_type=jax.ShapeDtypeStruct((n_tokens, C), jnp.float32),
        scratch_types=[
            pltpu.VMEM((t_per_w,), jnp.float32),
            pltpu.VMEM((t_per_w,), jnp.float32),
            pltpu.VMEM((t_per_w,), jnp.float32),
            pltpu.VMEM((t_per_w,), jnp.float32),
            pltpu.VMEM((16,), jnp.int32),
            pltpu.VMEM((16,), jnp.int32),
            pltpu.VMEM((16, C), jnp.float32),
            pltpu.VMEM((16, C), jnp.float32),
            pltpu.VMEM((16, C), jnp.float32),
            pltpu.SemaphoreType.DMA,
        ],
    )
    def combine(yg, dest, wn, outf,
                d0_v, d1_v, w0_v, w1_v, i0, i1,
                rows0, rows1, srows, sem):
        wid = lax.axis_index("s") * info.num_cores + lax.axis_index("c")
        tbase = pl.multiple_of(wid * t_per_w, 64)
        a0base = pl.multiple_of(((tbase >> 10) << 11) + (tbase & 1023), 64)
        pltpu.sync_copy(dest.at[pl.ds(a0base, t_per_w)], d0_v)
        pltpu.sync_copy(dest.at[pl.ds(a0base + 1024, t_per_w)], d1_v)
        pltpu.sync_copy(wn.at[pl.ds(a0base, t_per_w)], w0_v)
        pltpu.sync_copy(wn.at[pl.ds(a0base + 1024, t_per_w)], w1_v)

        for ch in range(n_ch):
            wv0 = w0_v[pl.ds(ch * 16, 16)]
            wv1 = w1_v[pl.ds(ch * 16, 16)]
            i0[...] = d0_v[pl.ds(ch * 16, 16)].astype(jnp.int32)
            i1[...] = d1_v[pl.ds(ch * 16, 16)].astype(jnp.int32)
            pltpu.async_copy(yg.at[i0], rows0, sem).wait()
            pltpu.async_copy(yg.at[i1], rows1, sem).wait()
            pltpu.sync_copy(
                yg.at[pl.ds(routed_pad + tbase + ch * 16, 16)], srows)
            for t in range(16):
                w0s = wv0[t]
                w1s = wv1[t]

                def body(j, _):
                    sl = pl.ds(j * 16, 16)
                    rows0[t, sl] = (w0s * rows0[t, sl] + w1s * rows1[t, sl]
                                    + srows[t, sl])
                    return 0

                lax.fori_loop(0, C // 16, body, 0)
            pltpu.sync_copy(rows0, outf.at[pl.ds(tbase + ch * 16, 16)])

    return combine


def kernel(x, Wg, W1, W2, W3, Ws1, Ws2, Ws3):
    Bb, Tt, C = x.shape
    n_tokens = Bb * Tt
    n_asn = 2 * n_tokens
    Hd = W1.shape[-1]
    Hp = ((Hd + 127) // 128) * 128
    x_flat = x.reshape(n_tokens, C)

    routed_pad = n_asn + N_EXP * BLK
    nbr = routed_pad // BLK
    nblk = nbr + n_tokens // BLK
    nblk_pad = nblk

    w1 = jnp.concatenate([W1, Ws1], axis=0)
    w2 = jnp.concatenate([W2, Ws2], axis=0)
    w3 = jnp.concatenate([W3, Ws3], axis=0)
    w1 = jnp.pad(w1, ((0, 0), (0, 0), (0, Hp - Hd))).astype(jnp.bfloat16)
    w2 = jnp.pad(w2, ((0, 0), (0, 0), (0, Hp - Hd))).astype(jnp.bfloat16)
    w3 = jnp.pad(w3, ((0, 0), (0, Hp - Hd), (0, 0))).astype(jnp.bfloat16)

    e_arr, w_arr, rank, segend, aux = _run_router(x_flat, Wg, n_tokens, C)
    wn = w_arr.reshape(-1)

    dest2, blk_e2 = _run_finalize(e_arr, rank, segend, n_asn, nbr, nblk_pad)
    dest = dest2.reshape(-1)
    blk_e = blk_e2.reshape(-1)

    dispatch = _make_dispatch(n_tokens, C, routed_pad)
    xg = dispatch(x_flat, dest)

    yg = _run_gmm(blk_e, xg, x_flat, w1, w2, w3, nbr, nblk, C, Hp)

    combine = _make_combine(n_tokens, C, routed_pad)
    outf = combine(yg, dest, wn)

    return (outf.reshape(Bb, Tt, C), aux[0, 0])


# speedup vs baseline: 1.0370x; 1.0370x over previous
import functools

import jax
import jax.numpy as jnp
from jax import lax
from jax.experimental import pallas as pl
from jax.experimental.pallas import tpu as pltpu
from jax.experimental.pallas import tpu_sc as plsc

N_EXP = 8
BLK = 256


def _router_body(x_ref, wg_ref, e_ref, w_ref, rank_ref, cnt_ref,
                 aux_ref, run_ref, acc_ref, *, n_tb, n_tokens, tb_sz):
    tb = pl.program_id(0)
    xb = x_ref[...]

    logits = jnp.dot(xb.astype(jnp.bfloat16),
                     wg_ref[...].astype(jnp.bfloat16),
                     preferred_element_type=jnp.float32)
    m = jnp.max(logits, axis=-1, keepdims=True)
    ex = jnp.exp(logits - m)
    p = ex / jnp.sum(ex, axis=-1, keepdims=True)
    lane = jax.lax.broadcasted_iota(jnp.int32, p.shape, 1).astype(jnp.float32)
    p1 = jnp.max(p, axis=-1, keepdims=True)
    i1 = jnp.min(jnp.where(p == p1, lane, float(N_EXP)), axis=-1,
                 keepdims=True)
    pm = jnp.where(lane == i1, -1.0, p)
    p2 = jnp.max(pm, axis=-1, keepdims=True)
    i2 = jnp.min(jnp.where(pm == p2, lane, float(N_EXP)), axis=-1,
                 keepdims=True)
    s = p1 + p2
    w1n = p1 / s
    w2n = p2 / s

    cnt_blk = jnp.sum(
        jnp.where(lane == i1, 1.0, 0.0) + jnp.where(lane == i2, 1.0, 0.0),
        axis=0, keepdims=True)
    sp = jnp.sum(p, axis=0, keepdims=True)
    new = jnp.concatenate([cnt_blk, sp], axis=0)

    @pl.when(tb == 0)
    def _():
        acc_ref[...] = new
        run_ref[...] = jnp.zeros_like(run_ref)

    @pl.when(tb > 0)
    def _():
        acc_ref[...] = acc_ref[...] + new

    @pl.when(tb == n_tb - 1)
    def _():
        a = acc_ref[...]
        aux_ref[...] = (N_EXP / (n_tokens * n_tokens)) * jnp.sum(
            a[0:1, :] * a[1:2, :], axis=-1, keepdims=True)

    e_flat = jnp.concatenate([i1, i2], axis=0)
    w_flat = jnp.concatenate([w1n, w2n], axis=0)
    e_ref[...] = e_flat
    w_ref[...] = w_flat

    lane2 = jax.lax.broadcasted_iota(
        jnp.int32, (2 * tb_sz, N_EXP), 1).astype(jnp.float32)
    oh = (lane2 == e_flat).astype(jnp.bfloat16)
    r = jax.lax.broadcasted_iota(jnp.int32, (2 * tb_sz, 2 * tb_sz), 0)
    c = jax.lax.broadcasted_iota(jnp.int32, (2 * tb_sz, 2 * tb_sz), 1)
    ltri = (c < r).astype(jnp.bfloat16)
    rank_in_blk = jnp.dot(ltri, oh, preferred_element_type=jnp.float32)
    rank_e = jnp.sum(rank_in_blk * oh.astype(jnp.float32), axis=-1,
                     keepdims=True)
    base = jnp.sum(oh.astype(jnp.float32) * run_ref[...], axis=-1,
                   keepdims=True)
    rank_ref[...] = base + rank_e
    run_ref[...] = run_ref[...] + cnt_blk

    @pl.when(tb == n_tb - 1)
    def _():
        cnt = run_ref[...]
        cap = jnp.ceil(cnt / BLK) * BLK
        r8 = jax.lax.broadcasted_iota(jnp.int32, (N_EXP, N_EXP), 0)
        c8 = jax.lax.broadcasted_iota(jnp.int32, (N_EXP, N_EXP), 1)
        ut = (r8 <= c8).astype(jnp.float32)
        cum = jnp.dot(cap, ut, preferred_element_type=jnp.float32)
        cnt_ref[...] = jnp.concatenate([cum - cap, cum], axis=1)


def _run_router(x_flat, Wg, n_tokens, C):
    TB = 1024
    n_tb = n_tokens // TB
    body = functools.partial(_router_body, n_tb=n_tb, n_tokens=n_tokens,
                             tb_sz=TB)
    return pl.pallas_call(
        body,
        grid=(n_tb,),
        in_specs=[
            pl.BlockSpec((TB, C), lambda tb: (tb, 0)),
            pl.BlockSpec((C, N_EXP), lambda tb: (0, 0)),
        ],
        out_specs=[
            pl.BlockSpec((2 * TB, 1), lambda tb: (tb, 0)),
            pl.BlockSpec((2 * TB, 1), lambda tb: (tb, 0)),
            pl.BlockSpec((2 * TB, 1), lambda tb: (tb, 0)),
            pl.BlockSpec((1, 2 * N_EXP), lambda tb: (0, 0)),
            pl.BlockSpec((1, 1), lambda tb: (0, 0)),
        ],
        out_shape=[
            jax.ShapeDtypeStruct((2 * n_tokens, 1), jnp.float32),
            jax.ShapeDtypeStruct((2 * n_tokens, 1), jnp.float32),
            jax.ShapeDtypeStruct((2 * n_tokens, 1), jnp.float32),
            jax.ShapeDtypeStruct((1, 2 * N_EXP), jnp.float32),
            jax.ShapeDtypeStruct((1, 1), jnp.float32),
        ],
        scratch_shapes=[
            pltpu.VMEM((1, N_EXP), jnp.float32),
            pltpu.VMEM((2, N_EXP), jnp.float32),
        ],
        compiler_params=pltpu.CompilerParams(
            dimension_semantics=("arbitrary",),
        ),
    )(x_flat, Wg)


def _finalize_body(e_in_ref, rank_ref, segend_ref, dest_ref, blk_ref,
                   *, nbr, nblk_pad):
    tb = pl.program_id(0)
    seg_f = segend_ref[...]
    e = e_in_ref[...]
    lane = jax.lax.broadcasted_iota(
        jnp.int32, (e.shape[0], N_EXP), 1).astype(jnp.float32)
    oh = (lane == e).astype(jnp.float32)
    start = jnp.sum(oh * seg_f[:, :N_EXP], axis=-1, keepdims=True)
    dest_ref[...] = start + rank_ref[...]

    @pl.when(tb == 0)
    def _():
        ends = seg_f[:, N_EXP:]
        bv = jax.lax.broadcasted_iota(
            jnp.int32, (nblk_pad, N_EXP), 0).astype(jnp.float32)
        acc = jnp.sum((bv * BLK >= ends).astype(jnp.int32),
                      axis=-1, keepdims=True)
        blk_ref[...] = jnp.minimum(acc, N_EXP - 1)


def _run_finalize(e_arr, rank, segend, n_asn, nbr, nblk_pad):
    AB = 2048
    return pl.pallas_call(
        functools.partial(_finalize_body, nbr=nbr, nblk_pad=nblk_pad),
        grid=(n_asn // AB,),
        in_specs=[
            pl.BlockSpec((AB, 1), lambda tb: (tb, 0)),
            pl.BlockSpec((AB, 1), lambda tb: (tb, 0)),
            pl.BlockSpec((1, 2 * N_EXP), lambda tb: (0, 0)),
        ],
        out_specs=[
            pl.BlockSpec((AB, 1), lambda tb: (tb, 0)),
            pl.BlockSpec((nblk_pad, 1), lambda tb: (0, 0)),
        ],
        out_shape=[
            jax.ShapeDtypeStruct((n_asn, 1), jnp.float32),
            jax.ShapeDtypeStruct((nblk_pad, 1), jnp.int32),
        ],
        compiler_params=pltpu.CompilerParams(
            dimension_semantics=("arbitrary",),
        ),
    )(e_arr, rank, segend)


def _make_dispatch(n_tokens, C, routed_pad):
    n_asn = 2 * n_tokens
    info = plsc.get_sparse_core_info()
    nw = info.num_cores * info.num_subcores
    a_per_w = n_asn // nw
    n_ch = a_per_w // 64
    mesh = plsc.VectorSubcoreMesh(core_axis_name="c", subcore_axis_name="s")

    @functools.partial(
        pl.kernel, mesh=mesh,
        out_type=jax.ShapeDtypeStruct((routed_pad, C), jnp.float32),
        scratch_types=[
            pltpu.VMEM((a_per_w,), jnp.float32),
            pltpu.VMEM((64,), jnp.int32),
            pltpu.VMEM((64,), jnp.int32),
            pltpu.VMEM((64, C), jnp.float32),
            pltpu.SemaphoreType.DMA,
        ],
    )
    def dispatch(xf32, dest, xg, dest_v, tok_i, dst_i, rows, sem):
        wid = lax.axis_index("s") * info.num_cores + lax.axis_index("c")
        abase = pl.multiple_of(wid * a_per_w, 64)
        l16 = lax.iota(jnp.int32, 16)

        pltpu.sync_copy(dest.at[pl.ds(abase, a_per_w)], dest_v)
        for ch in range(n_ch):
            for j in range(4):
                a_v = abase + ch * 64 + j * 16 + l16
                t_v = ((a_v >> 11) << 10) + (a_v & 1023)
                tok_i[pl.ds(j * 16, 16)] = t_v
                dst_i[pl.ds(j * 16, 16)] = dest_v[
                    pl.ds(ch * 64 + j * 16, 16)].astype(jnp.int32)
            pltpu.async_copy(xf32.at[tok_i], rows, sem).wait()
            pltpu.async_copy(rows, xg.at[dst_i], sem).wait()

    return dispatch


def _gmm_body(s_ref, xg_ref, w1_ref, w2_ref, w3_ref, yg_ref):
    g = jnp.dot(xg_ref[...].astype(jnp.bfloat16), w1_ref[0],
                preferred_element_type=jnp.float32)
    u = jnp.dot(xg_ref[...].astype(jnp.bfloat16), w2_ref[0],
                preferred_element_type=jnp.float32)
    act = (g * jax.nn.sigmoid(g) * u).astype(jnp.bfloat16)
    yg_ref[...] = jnp.dot(act, w3_ref[0], preferred_element_type=jnp.float32)


def _run_gmm(blk_e, xg, w1, w2, w3, nbr, C, Hd):
    grid_spec = pltpu.PrefetchScalarGridSpec(
        num_scalar_prefetch=1,
        grid=(nbr,),
        in_specs=[
            pl.BlockSpec((BLK, C), lambda b, s: (b, 0)),
            pl.BlockSpec((1, C, Hd), lambda b, s: (s[b], 0, 0)),
            pl.BlockSpec((1, C, Hd), lambda b, s: (s[b], 0, 0)),
            pl.BlockSpec((1, Hd, C), lambda b, s: (s[b], 0, 0)),
        ],
        out_specs=pl.BlockSpec((BLK, C), lambda b, s: (b, 0)),
    )
    return pl.pallas_call(
        _gmm_body,
        grid_spec=grid_spec,
        out_shape=jax.ShapeDtypeStruct((nbr * BLK, C), jnp.float32),
        compiler_params=pltpu.CompilerParams(
            dimension_semantics=("arbitrary",),
        ),
    )(blk_e, xg, w1, w2, w3)


def _shared_body(x_ref, w1_ref, w2_ref, w3_ref, ys_ref):
    xb = x_ref[...].astype(jnp.bfloat16)
    g = jnp.dot(xb, w1_ref[0], preferred_element_type=jnp.float32)
    u = jnp.dot(xb, w2_ref[0], preferred_element_type=jnp.float32)
    act = (g * jax.nn.sigmoid(g) * u).astype(jnp.bfloat16)
    ys_ref[...] = jnp.dot(act, w3_ref[0], preferred_element_type=jnp.float32)


def _run_shared(x_flat, w1, w2, w3, n_tokens, C, Hd):
    SB = 1024
    return pl.pallas_call(
        _shared_body,
        grid=(n_tokens // SB,),
        in_specs=[
            pl.BlockSpec((SB, C), lambda b: (b, 0)),
            pl.BlockSpec((1, C, Hd), lambda b: (0, 0, 0)),
            pl.BlockSpec((1, C, Hd), lambda b: (0, 0, 0)),
            pl.BlockSpec((1, Hd, C), lambda b: (0, 0, 0)),
        ],
        out_specs=pl.BlockSpec((SB, C), lambda b: (b, 0)),
        out_shape=jax.ShapeDtypeStruct((n_tokens, C), jnp.float32),
        compiler_params=pltpu.CompilerParams(
            dimension_semantics=("arbitrary",),
        ),
    )(x_flat, w1, w2, w3)


def _make_combine(n_tokens, C):
    info = plsc.get_sparse_core_info()
    nw = info.num_cores * info.num_subcores
    t_per_w = n_tokens // nw
    n_ch = t_per_w // 16
    mesh = plsc.VectorSubcoreMesh(core_axis_name="c", subcore_axis_name="s")

    @functools.partial(
        pl.kernel, mesh=mesh,
        out_type=jax.ShapeDtypeStruct((n_tokens, C), jnp.float32),
        scratch_types=[
            pltpu.VMEM((t_per_w,), jnp.float32),
            pltpu.VMEM((t_per_w,), jnp.float32),
            pltpu.VMEM((t_per_w,), jnp.float32),
            pltpu.VMEM((t_per_w,), jnp.float32),
            pltpu.VMEM((2, 16), jnp.int32),
            pltpu.VMEM((2, 16), jnp.int32),
            pltpu.VMEM((2, 16, C), jnp.float32),
            pltpu.VMEM((2, 16, C), jnp.float32),
            pltpu.VMEM((2, 16, C), jnp.float32),
            pltpu.SemaphoreType.DMA,
            pltpu.SemaphoreType.DMA,
        ],
    )
    def combine(ygr, ygs, dest, wn, outf,
                d0_v, d1_v, w0_v, w1_v, i0, i1,
                rows0, rows1, srows, sem, sem_o):
        wid = lax.axis_index("s") * info.num_cores + lax.axis_index("c")
        tbase = pl.multiple_of(wid * t_per_w, 64)
        a0base = pl.multiple_of(((tbase >> 10) << 11) + (tbase & 1023), 64)
        pltpu.sync_copy(dest.at[pl.ds(a0base, t_per_w)], d0_v)
        pltpu.sync_copy(dest.at[pl.ds(a0base + 1024, t_per_w)], d1_v)
        pltpu.sync_copy(wn.at[pl.ds(a0base, t_per_w)], w0_v)
        pltpu.sync_copy(wn.at[pl.ds(a0base + 1024, t_per_w)], w1_v)

        def fire(ch):
            p = ch & 1
            i0[p, :] = d0_v[pl.ds(ch * 16, 16)].astype(jnp.int32)
            i1[p, :] = d1_v[pl.ds(ch * 16, 16)].astype(jnp.int32)
            return (
                pltpu.async_copy(ygr.at[i0.at[p]], rows0.at[p], sem),
                pltpu.async_copy(ygr.at[i1.at[p]], rows1.at[p], sem),
                pltpu.async_copy(ygs.at[pl.ds(tbase + ch * 16, 16)],
                                 srows.at[p], sem),
            )

        pend = fire(0)
        out_q = [None, None]
        for ch in range(n_ch):
            p = ch & 1
            cur = pend
            if ch + 1 < n_ch:
                q = (ch + 1) & 1
                if out_q[q] is not None:
                    out_q[q].wait()
                    out_q[q] = None
                pend = fire(ch + 1)
            for cp in cur:
                cp.wait()
            wv0 = w0_v[pl.ds(ch * 16, 16)]
            wv1 = w1_v[pl.ds(ch * 16, 16)]
            for t in range(16):
                w0s = wv0[t]
                w1s = wv1[t]

                def body(j, _):
                    sl = pl.ds(j * 16, 16)
                    rows0[p, t, sl] = (w0s * rows0[p, t, sl]
                                       + w1s * rows1[p, t, sl]
                                       + srows[p, t, sl])
                    return 0

                lax.fori_loop(0, C // 16, body, 0)
            out_q[p] = pltpu.async_copy(
                rows0.at[p], outf.at[pl.ds(tbase + ch * 16, 16)], sem_o)
        for oc in out_q:
            if oc is not None:
                oc.wait()

    return combine


def kernel(x, Wg, W1, W2, W3, Ws1, Ws2, Ws3):
    Bb, Tt, C = x.shape
    n_tokens = Bb * Tt
    n_asn = 2 * n_tokens
    Hd = W1.shape[-1]
    x_flat = x.reshape(n_tokens, C)

    routed_pad = n_asn + N_EXP * BLK
    nbr = routed_pad // BLK

    w1 = W1.astype(jnp.bfloat16)
    w2 = W2.astype(jnp.bfloat16)
    w3 = W3.astype(jnp.bfloat16)
    ws1 = Ws1.astype(jnp.bfloat16)
    ws2 = Ws2.astype(jnp.bfloat16)
    ws3 = Ws3.astype(jnp.bfloat16)

    e_arr, w_arr, rank, segend, aux = _run_router(x_flat, Wg, n_tokens, C)
    wn = w_arr.reshape(-1)

    dest2, blk_e2 = _run_finalize(e_arr, rank, segend, n_asn, nbr, nbr)
    dest = dest2.reshape(-1)
    blk_e = blk_e2.reshape(-1)

    dispatch = _make_dispatch(n_tokens, C, routed_pad)
    xg = dispatch(x_flat, dest)

    ygs = _run_shared(x_flat, ws1, ws2, ws3, n_tokens, C, Hd)
    ygr = _run_gmm(blk_e, xg, w1, w2, w3, nbr, C, Hd)

    combine = _make_combine(n_tokens, C)
    outf = combine(ygr, ygs, dest, wn)

    return (outf.reshape(Bb, Tt, C), aux[0, 0])


# speedup vs baseline: 1.1403x; 1.0995x over previous
import functools

import jax
import jax.numpy as jnp
from jax import lax
from jax.experimental import pallas as pl
from jax.experimental.pallas import tpu as pltpu
from jax.experimental.pallas import tpu_sc as plsc

N_EXP = 8
BLK = 256


def _router_body(x_ref, wg_ref, e_ref, w_ref, rank_ref, cnt_ref,
                 aux_ref, run_ref, acc_ref, *, n_tb, n_tokens, tb_sz):
    tb = pl.program_id(0)
    xb = x_ref[...]

    logits = jnp.dot(xb.astype(jnp.bfloat16),
                     wg_ref[...].astype(jnp.bfloat16),
                     preferred_element_type=jnp.float32)
    m = jnp.max(logits, axis=-1, keepdims=True)
    ex = jnp.exp(logits - m)
    p = ex / jnp.sum(ex, axis=-1, keepdims=True)
    lane = jax.lax.broadcasted_iota(jnp.int32, p.shape, 1).astype(jnp.float32)
    p1 = jnp.max(p, axis=-1, keepdims=True)
    i1 = jnp.min(jnp.where(p == p1, lane, float(N_EXP)), axis=-1,
                 keepdims=True)
    pm = jnp.where(lane == i1, -1.0, p)
    p2 = jnp.max(pm, axis=-1, keepdims=True)
    i2 = jnp.min(jnp.where(pm == p2, lane, float(N_EXP)), axis=-1,
                 keepdims=True)
    s = p1 + p2
    w1n = p1 / s
    w2n = p2 / s

    cnt_blk = jnp.sum(
        jnp.where(lane == i1, 1.0, 0.0) + jnp.where(lane == i2, 1.0, 0.0),
        axis=0, keepdims=True)
    sp = jnp.sum(p, axis=0, keepdims=True)
    new = jnp.concatenate([cnt_blk, sp], axis=0)

    @pl.when(tb == 0)
    def _():
        acc_ref[...] = new
        run_ref[...] = jnp.zeros_like(run_ref)

    @pl.when(tb > 0)
    def _():
        acc_ref[...] = acc_ref[...] + new

    @pl.when(tb == n_tb - 1)
    def _():
        a = acc_ref[...]
        aux_ref[...] = (N_EXP / (n_tokens * n_tokens)) * jnp.sum(
            a[0:1, :] * a[1:2, :], axis=-1, keepdims=True)

    e_flat = jnp.concatenate([i1, i2], axis=0)
    w_flat = jnp.concatenate([w1n, w2n], axis=0)
    e_ref[...] = e_flat
    w_ref[...] = w_flat

    lane2 = jax.lax.broadcasted_iota(
        jnp.int32, (2 * tb_sz, N_EXP), 1).astype(jnp.float32)
    oh = (lane2 == e_flat).astype(jnp.bfloat16)
    r = jax.lax.broadcasted_iota(jnp.int32, (2 * tb_sz, 2 * tb_sz), 0)
    c = jax.lax.broadcasted_iota(jnp.int32, (2 * tb_sz, 2 * tb_sz), 1)
    ltri = (c < r).astype(jnp.bfloat16)
    rank_in_blk = jnp.dot(ltri, oh, preferred_element_type=jnp.float32)
    rank_e = jnp.sum(rank_in_blk * oh.astype(jnp.float32), axis=-1,
                     keepdims=True)
    base = jnp.sum(oh.astype(jnp.float32) * run_ref[...], axis=-1,
                   keepdims=True)
    rank_ref[...] = base + rank_e
    run_ref[...] = run_ref[...] + cnt_blk

    @pl.when(tb == n_tb - 1)
    def _():
        cnt = run_ref[...]
        cap = jnp.ceil(cnt / BLK) * BLK
        r8 = jax.lax.broadcasted_iota(jnp.int32, (N_EXP, N_EXP), 0)
        c8 = jax.lax.broadcasted_iota(jnp.int32, (N_EXP, N_EXP), 1)
        ut = (r8 <= c8).astype(jnp.float32)
        cum = jnp.dot(cap, ut, preferred_element_type=jnp.float32)
        cnt_ref[...] = jnp.concatenate([cum - cap, cum], axis=1)


def _run_router(x_flat, Wg, n_tokens, C):
    TB = 1024
    n_tb = n_tokens // TB
    body = functools.partial(_router_body, n_tb=n_tb, n_tokens=n_tokens,
                             tb_sz=TB)
    return pl.pallas_call(
        body,
        grid=(n_tb,),
        in_specs=[
            pl.BlockSpec((TB, C), lambda tb: (tb, 0)),
            pl.BlockSpec((C, N_EXP), lambda tb: (0, 0)),
        ],
        out_specs=[
            pl.BlockSpec((2 * TB, 1), lambda tb: (tb, 0)),
            pl.BlockSpec((2 * TB, 1), lambda tb: (tb, 0)),
            pl.BlockSpec((2 * TB, 1), lambda tb: (tb, 0)),
            pl.BlockSpec((1, 2 * N_EXP), lambda tb: (0, 0)),
            pl.BlockSpec((1, 1), lambda tb: (0, 0)),
        ],
        out_shape=[
            jax.ShapeDtypeStruct((2 * n_tokens, 1), jnp.float32),
            jax.ShapeDtypeStruct((2 * n_tokens, 1), jnp.float32),
            jax.ShapeDtypeStruct((2 * n_tokens, 1), jnp.float32),
            jax.ShapeDtypeStruct((1, 2 * N_EXP), jnp.float32),
            jax.ShapeDtypeStruct((1, 1), jnp.float32),
        ],
        scratch_shapes=[
            pltpu.VMEM((1, N_EXP), jnp.float32),
            pltpu.VMEM((2, N_EXP), jnp.float32),
        ],
        compiler_params=pltpu.CompilerParams(
            dimension_semantics=("arbitrary",),
        ),
    )(x_flat, Wg)


def _finalize_body(e_in_ref, rank_ref, segend_ref, dest_ref, blk_ref,
                   *, nbr, nblk_pad):
    tb = pl.program_id(0)
    seg_f = segend_ref[...]
    e = e_in_ref[...]
    lane = jax.lax.broadcasted_iota(
        jnp.int32, (e.shape[0], N_EXP), 1).astype(jnp.float32)
    oh = (lane == e).astype(jnp.float32)
    start = jnp.sum(oh * seg_f[:, :N_EXP], axis=-1, keepdims=True)
    dest_ref[...] = start + rank_ref[...]

    @pl.when(tb == 0)
    def _():
        ends = seg_f[:, N_EXP:]
        bv = jax.lax.broadcasted_iota(
            jnp.int32, (nblk_pad, N_EXP), 0).astype(jnp.float32)
        acc = jnp.sum((bv * BLK >= ends).astype(jnp.int32),
                      axis=-1, keepdims=True)
        blk_ref[...] = jnp.minimum(acc, N_EXP - 1)


def _run_finalize(e_arr, rank, segend, n_asn, nbr, nblk_pad):
    AB = 2048
    return pl.pallas_call(
        functools.partial(_finalize_body, nbr=nbr, nblk_pad=nblk_pad),
        grid=(n_asn // AB,),
        in_specs=[
            pl.BlockSpec((AB, 1), lambda tb: (tb, 0)),
            pl.BlockSpec((AB, 1), lambda tb: (tb, 0)),
            pl.BlockSpec((1, 2 * N_EXP), lambda tb: (0, 0)),
        ],
        out_specs=[
            pl.BlockSpec((AB, 1), lambda tb: (tb, 0)),
            pl.BlockSpec((nblk_pad, 1), lambda tb: (0, 0)),
        ],
        out_shape=[
            jax.ShapeDtypeStruct((n_asn, 1), jnp.float32),
            jax.ShapeDtypeStruct((nblk_pad, 1), jnp.int32),
        ],
        compiler_params=pltpu.CompilerParams(
            dimension_semantics=("arbitrary",),
        ),
    )(e_arr, rank, segend)


def _make_dispatch(n_tokens, C, routed_pad):
    n_asn = 2 * n_tokens
    info = plsc.get_sparse_core_info()
    nw = info.num_cores * info.num_subcores
    a_per_w = n_asn // nw
    n_ch = a_per_w // 64
    mesh = plsc.VectorSubcoreMesh(core_axis_name="c", subcore_axis_name="s")

    @functools.partial(
        pl.kernel, mesh=mesh,
        out_type=jax.ShapeDtypeStruct((routed_pad, C), jnp.float32),
        scratch_types=[
            pltpu.VMEM((a_per_w,), jnp.float32),
            pltpu.VMEM((64,), jnp.int32),
            pltpu.VMEM((64,), jnp.int32),
            pltpu.VMEM((64, C), jnp.float32),
            pltpu.SemaphoreType.DMA,
        ],
    )
    def dispatch(xf32, dest, xg, dest_v, tok_i, dst_i, rows, sem):
        wid = lax.axis_index("s") * info.num_cores + lax.axis_index("c")
        abase = pl.multiple_of(wid * a_per_w, 64)
        l16 = lax.iota(jnp.int32, 16)

        pltpu.sync_copy(dest.at[pl.ds(abase, a_per_w)], dest_v)
        for ch in range(n_ch):
            for j in range(4):
                a_v = abase + ch * 64 + j * 16 + l16
                t_v = ((a_v >> 11) << 10) + (a_v & 1023)
                tok_i[pl.ds(j * 16, 16)] = t_v
                dst_i[pl.ds(j * 16, 16)] = dest_v[
                    pl.ds(ch * 64 + j * 16, 16)].astype(jnp.int32)
            pltpu.async_copy(xf32.at[tok_i], rows, sem).wait()
            pltpu.async_copy(rows, xg.at[dst_i], sem).wait()

    return dispatch


def _gmm_body(s_ref, xg_ref, w1_ref, w2_ref, w3_ref, yg_ref):
    xb = xg_ref[...].astype(jnp.bfloat16)
    g = jnp.dot(xb, w1_ref[0], preferred_element_type=jnp.float32)
    u = jnp.dot(xb, w2_ref[0], preferred_element_type=jnp.float32)
    act = (g * jax.nn.sigmoid(g) * u).astype(jnp.bfloat16)
    yg_ref[...] = jnp.dot(act, w3_ref[0], preferred_element_type=jnp.float32)


def _run_gmm(blk_e, xg, w1, w2, w3, nbr, C, Hd):
    grid_spec = pltpu.PrefetchScalarGridSpec(
        num_scalar_prefetch=1,
        grid=(nbr,),
        in_specs=[
            pl.BlockSpec((BLK, C), lambda b, s: (b, 0)),
            pl.BlockSpec((1, C, Hd), lambda b, s: (s[b], 0, 0)),
            pl.BlockSpec((1, C, Hd), lambda b, s: (s[b], 0, 0)),
            pl.BlockSpec((1, Hd, C), lambda b, s: (s[b], 0, 0)),
        ],
        out_specs=pl.BlockSpec((BLK, C), lambda b, s: (b, 0)),
    )
    return pl.pallas_call(
        _gmm_body,
        grid_spec=grid_spec,
        out_shape=jax.ShapeDtypeStruct((nbr * BLK, C), jnp.float32),
        compiler_params=pltpu.CompilerParams(
            dimension_semantics=("arbitrary",),
        ),
    )(blk_e, xg, w1, w2, w3)


def _shared_body(x_ref, w1_ref, w2_ref, w3_ref, out_ref):
    xb = x_ref[...].astype(jnp.bfloat16)
    g = jnp.dot(xb, w1_ref[0], preferred_element_type=jnp.float32)
    u = jnp.dot(xb, w2_ref[0], preferred_element_type=jnp.float32)
    act = (g * jax.nn.sigmoid(g) * u).astype(jnp.bfloat16)
    out_ref[...] = jnp.dot(act, w3_ref[0], preferred_element_type=jnp.float32)


def _run_shared(x_flat, w1, w2, w3, n_tokens, C, Hd):
    SB = 1024
    return pl.pallas_call(
        _shared_body,
        grid=(n_tokens // SB,),
        in_specs=[
            pl.BlockSpec((SB, C), lambda b: (b, 0)),
            pl.BlockSpec((1, C, Hd), lambda b: (0, 0, 0)),
            pl.BlockSpec((1, C, Hd), lambda b: (0, 0, 0)),
            pl.BlockSpec((1, Hd, C), lambda b: (0, 0, 0)),
        ],
        out_specs=pl.BlockSpec((SB, C), lambda b: (b, 0)),
        out_shape=jax.ShapeDtypeStruct((n_tokens, C), jnp.float32),
        compiler_params=pltpu.CompilerParams(
            dimension_semantics=("arbitrary",),
        ),
    )(x_flat, w1, w2, w3)


def _add_body(a_ref, b_ref, o_ref):
    o_ref[...] = a_ref[...] + b_ref[...]


def _run_add(a, b, n_tokens, C):
    SB = 2048
    return pl.pallas_call(
        _add_body,
        grid=(n_tokens // SB,),
        in_specs=[
            pl.BlockSpec((SB, C), lambda i: (i, 0)),
            pl.BlockSpec((SB, C), lambda i: (i, 0)),
        ],
        out_specs=pl.BlockSpec((SB, C), lambda i: (i, 0)),
        out_shape=jax.ShapeDtypeStruct((n_tokens, C), jnp.float32),
        compiler_params=pltpu.CompilerParams(
            dimension_semantics=("arbitrary",),
        ),
    )(a, b)


def _make_combine(n_tokens, C):
    info = plsc.get_sparse_core_info()
    nw = info.num_cores * info.num_subcores
    t_per_w = n_tokens // nw
    n_ch = t_per_w // 16
    mesh = plsc.VectorSubcoreMesh(core_axis_name="c", subcore_axis_name="s")

    @functools.partial(
        pl.kernel, mesh=mesh,
        out_type=jax.ShapeDtypeStruct((n_tokens, C), jnp.float32),
        scratch_types=[
            pltpu.VMEM((t_per_w,), jnp.float32),
            pltpu.VMEM((t_per_w,), jnp.float32),
            pltpu.VMEM((t_per_w,), jnp.float32),
            pltpu.VMEM((t_per_w,), jnp.float32),
            pltpu.VMEM((2, 16), jnp.int32),
            pltpu.VMEM((2, 16), jnp.int32),
            pltpu.VMEM((2, 16, C), jnp.float32),
            pltpu.VMEM((2, 16, C), jnp.float32),
            pltpu.SemaphoreType.DMA,
            pltpu.SemaphoreType.DMA,
        ],
    )
    def combine(ygr, dest, wn, outf,
                d0_v, d1_v, w0_v, w1_v, i0, i1,
                rows0, rows1, sem, sem_o):
        wid = lax.axis_index("s") * info.num_cores + lax.axis_index("c")
        tbase = pl.multiple_of(wid * t_per_w, 64)
        a0base = pl.multiple_of(((tbase >> 10) << 11) + (tbase & 1023), 64)
        pltpu.sync_copy(dest.at[pl.ds(a0base, t_per_w)], d0_v)
        pltpu.sync_copy(dest.at[pl.ds(a0base + 1024, t_per_w)], d1_v)
        pltpu.sync_copy(wn.at[pl.ds(a0base, t_per_w)], w0_v)
        pltpu.sync_copy(wn.at[pl.ds(a0base + 1024, t_per_w)], w1_v)

        def fire(ch):
            p = ch & 1
            i0[p, :] = d0_v[pl.ds(ch * 16, 16)].astype(jnp.int32)
            i1[p, :] = d1_v[pl.ds(ch * 16, 16)].astype(jnp.int32)
            return (
                pltpu.async_copy(ygr.at[i0.at[p]], rows0.at[p], sem),
                pltpu.async_copy(ygr.at[i1.at[p]], rows1.at[p], sem),
            )

        pend = fire(0)
        out_q = [None, None]
        for ch in range(n_ch):
            p = ch & 1
            cur = pend
            if ch + 1 < n_ch:
                q = (ch + 1) & 1
                if out_q[q] is not None:
                    out_q[q].wait()
                    out_q[q] = None
                pend = fire(ch + 1)
            for cp in cur:
                cp.wait()
            wv0 = w0_v[pl.ds(ch * 16, 16)]
            wv1 = w1_v[pl.ds(ch * 16, 16)]
            for t in range(16):
                w0s = wv0[t]
                w1s = wv1[t]

                def body(j, _):
                    sl = pl.ds(j * 16, 16)
                    rows0[p, t, sl] = (w0s * rows0[p, t, sl]
                                       + w1s * rows1[p, t, sl])
                    return 0

                lax.fori_loop(0, C // 16, body, 0)
            out_q[p] = pltpu.async_copy(
                rows0.at[p], outf.at[pl.ds(tbase + ch * 16, 16)], sem_o)
        for oc in out_q:
            if oc is not None:
                oc.wait()

    return combine


def kernel(x, Wg, W1, W2, W3, Ws1, Ws2, Ws3):
    Bb, Tt, C = x.shape
    n_tokens = Bb * Tt
    n_asn = 2 * n_tokens
    Hd = W1.shape[-1]
    x_flat = x.reshape(n_tokens, C)

    routed_pad = n_asn + N_EXP * BLK
    nbr = routed_pad // BLK

    w1 = W1.astype(jnp.bfloat16)
    w2 = W2.astype(jnp.bfloat16)
    w3 = W3.astype(jnp.bfloat16)
    ws1 = Ws1.astype(jnp.bfloat16)
    ws2 = Ws2.astype(jnp.bfloat16)
    ws3 = Ws3.astype(jnp.bfloat16)

    e_arr, w_arr, rank, segend, aux = _run_router(x_flat, Wg, n_tokens, C)
    wn = w_arr.reshape(-1)

    dest2, blk_e2 = _run_finalize(e_arr, rank, segend, n_asn, nbr, nbr)
    dest = dest2.reshape(-1)
    blk_e = blk_e2.reshape(-1)

    dispatch = _make_dispatch(n_tokens, C, routed_pad)
    xg = dispatch(x_flat, dest)

    ygr = _run_gmm(blk_e, xg, w1, w2, w3, nbr, C, Hd)

    combine = _make_combine(n_tokens, C)
    rc = combine(ygr, dest, wn)

    ygs = _run_shared(x_flat, ws1, ws2, ws3, n_tokens, C, Hd)
    outf = _run_add(rc, ygs, n_tokens, C)

    return (outf.reshape(Bb, Tt, C), aux[0, 0])
